# baseline (speedup 1.0000x reference)
import functools

import jax
import jax.numpy as jnp
from jax import lax
from jax.experimental import pallas as pl
from jax.experimental.pallas import tpu as pltpu
from jax.experimental.pallas import tpu_sc as plsc

B = 16384
D = 16
L = 16
NC = 2
NS = 16
NW = NC * NS
BPW = B // NW
CH = 128
NCH = BPW // CH
NV = BPW // L

_mesh = plsc.VectorSubcoreMesh(core_axis_name="c", subcore_axis_name="s")


@functools.partial(
    pl.kernel,
    out_type=jax.ShapeDtypeStruct((B,), jnp.float32),
    mesh=_mesh,
    compiler_params=pltpu.CompilerParams(use_tc_tiling_on_sc=False),
    scratch_types=[
        pltpu.VMEM((BPW,), jnp.int32),
        pltpu.VMEM((BPW,), jnp.int32),
        pltpu.VMEM((D, BPW), jnp.float32),
        pltpu.VMEM((D, BPW), jnp.float32),
        pltpu.VMEM((BPW,), jnp.float32),
        pltpu.SemaphoreType.DMA,
        pltpu.SemaphoreType.DMA,
    ],
)
def _dot_kernel(u_hbm, i_hbm, fut_hbm, fit_hbm, out_hbm,
                uidx, iidx, ucols, icols, outv, usem, isem):
    wid = lax.axis_index("s") * NC + lax.axis_index("c")
    base = wid * BPW

    for c in range(NCH):
        pltpu.sync_copy(u_hbm.at[pl.ds(base + c * CH, CH)],
                        uidx.at[pl.ds(c * CH, CH)])
        pltpu.sync_copy(i_hbm.at[pl.ds(base + c * CH, CH)],
                        iidx.at[pl.ds(c * CH, CH)])

    for d in range(D):
        for c in range(NCH):
            pltpu.async_copy(
                fut_hbm.at[d].at[uidx.at[pl.ds(c * CH, CH)]],
                ucols.at[d, pl.ds(c * CH, CH)], usem)
            pltpu.async_copy(
                fit_hbm.at[d].at[iidx.at[pl.ds(c * CH, CH)]],
                icols.at[d, pl.ds(c * CH, CH)], isem)

    pltpu.make_async_copy(fut_hbm.at[:, pl.ds(0, BPW)], ucols, usem).wait()
    pltpu.make_async_copy(fit_hbm.at[:, pl.ds(0, BPW)], icols, isem).wait()

    def tile(t, carry):
        acc = ucols[0, pl.ds(t * L, L)] * icols[0, pl.ds(t * L, L)]
        for d in range(1, D):
            acc = acc + (ucols[d, pl.ds(t * L, L)]
                         * icols[d, pl.ds(t * L, L)])
        outv[pl.ds(t * L, L)] = acc
        return carry

    lax.fori_loop(0, NV, tile, 0)

    pltpu.sync_copy(outv, out_hbm.at[pl.ds(base, BPW)])


def kernel(u, i, feat_u, feat_i):
    return _dot_kernel(u, i, feat_u.T, feat_i.T)


# speedup vs baseline: 3.1988x; 3.1988x over previous
import functools

import jax
import jax.numpy as jnp
from jax import lax
from jax.experimental import pallas as pl
from jax.experimental.pallas import tpu as pltpu
from jax.experimental.pallas import tpu_sc as plsc

B = 16384
D = 16
L = 16
NC = 2
NS = 16
NW = NC * NS
BPW = B // NW
CH = 128
NCH = BPW // CH

_mesh = plsc.VectorSubcoreMesh(core_axis_name="c", subcore_axis_name="s")


@functools.partial(
    pl.kernel,
    out_type=jax.ShapeDtypeStruct((B,), jnp.float32),
    mesh=_mesh,
    compiler_params=pltpu.CompilerParams(use_tc_tiling_on_sc=False),
    scratch_types=[
        pltpu.VMEM((NCH, CH), jnp.int32),
        pltpu.VMEM((NCH, CH), jnp.int32),
        pltpu.VMEM((BPW, D), jnp.float32),
        pltpu.VMEM((BPW, D), jnp.float32),
        pltpu.VMEM((BPW,), jnp.float32),
        pltpu.SemaphoreType.DMA,
    ],
)
def _dot_kernel(u_hbm, i_hbm, fu_hbm, fi_hbm, out_hbm,
                uidx, iidx, urows, irows, outv, sem):
    wid = lax.axis_index("s") * NC + lax.axis_index("c")
    base = wid * BPW

    copies = []
    for j in range(NCH):
        pltpu.sync_copy(u_hbm.at[pl.ds(base + j * CH, CH)], uidx.at[j])
        copies.append(pltpu.async_copy(
            fu_hbm.at[uidx.at[j]], urows.at[pl.ds(j * CH, CH)], sem))
    for j in range(NCH):
        pltpu.sync_copy(i_hbm.at[pl.ds(base + j * CH, CH)], iidx.at[j])
        copies.append(pltpu.async_copy(
            fi_hbm.at[iidx.at[j]], irows.at[pl.ds(j * CH, CH)], sem))
    for c in copies:
        c.wait()

    lane = lax.iota(jnp.int32, L)

    def tile(t, carry):
        base_r = t * L
        acc = jnp.zeros((L,), jnp.float32)
        for k in range(L):
            prod = urows[base_r + k, :] * irows[base_r + k, :]
            for st in (1, 2, 4, 8):
                prod = prod + prod.at[lane ^ st].get(
                    mode="promise_in_bounds")
            acc = jnp.where(lane == k, prod, acc)
        outv[pl.ds(base_r, L)] = acc
        return carry

    lax.fori_loop(0, BPW // L, tile, 0)

    pltpu.sync_copy(outv, out_hbm.at[pl.ds(base, BPW)])


def kernel(u, i, feat_u, feat_i):
    return _dot_kernel(u, i, feat_u, feat_i)
